# baseline (speedup 1.0000x reference)
import jax
import jax.numpy as jnp
from jax.experimental import pallas as pl
from jax.experimental.pallas import tpu as pltpu


def _copy_body(pos_ref, out_ref, sem):
    copy = pltpu.make_async_copy(pos_ref, out_ref, sem)
    copy.start()
    copy.wait()


def kernel(x, pos_table):
    del x
    return pl.pallas_call(
        _copy_body,
        in_specs=[pl.BlockSpec(memory_space=pl.ANY)],
        out_specs=pl.BlockSpec(memory_space=pl.ANY),
        scratch_shapes=[pltpu.SemaphoreType.DMA],
        out_shape=jax.ShapeDtypeStruct(pos_table.shape, pos_table.dtype),
    )(pos_table)


# speedup vs baseline: 1.5730x; 1.5730x over previous
import jax
import jax.numpy as jnp
from jax.experimental import pallas as pl
from jax.experimental.pallas import tpu as pltpu


def _copy_body(pos_ref, out_ref):
    out_ref[...] = pos_ref[...]


def kernel(x, pos_table):
    del x
    return pl.pallas_call(
        _copy_body,
        out_shape=jax.ShapeDtypeStruct(pos_table.shape, pos_table.dtype),
    )(pos_table)


# speedup vs baseline: 4.9514x; 3.1478x over previous
import jax
import jax.numpy as jnp
from jax.experimental import pallas as pl


def _copy_body(pos_ref, out_ref):
    out_ref[...] = pos_ref[...]


def kernel(x, pos_table):
    del x
    t = pos_table.T
    out_t = pl.pallas_call(
        _copy_body,
        out_shape=jax.ShapeDtypeStruct(t.shape, t.dtype),
    )(t)
    return out_t.T
